# baseline (speedup 1.0000x reference)
import functools

import jax
import jax.numpy as jnp
from jax import lax
from jax.experimental import pallas as pl
from jax.experimental.pallas import tpu as pltpu
from jax.experimental.pallas import tpu_sc as plsc

D = 128
L = 512
CTX = 255
B = 1024
BL = B * L
NC, NS = 2, 16
NW = NC * NS
HALF = L // 2
BPW = B // (NW // 2)
C = 128
GC = BPW * HALF // C
NBUF = 4
LANES = 16
GROUPS = D // LANES


def _sc_body(seq_hbm, tok_hbm, pos_hbm, seg_hbm, out_hbm,
             idx_v, buf_v, bias_v, seg_v, gsems, osems):
    wid = lax.axis_index("s") * NC + lax.axis_index("c")
    h = wid % 2
    b0 = (wid // 2) * BPW

    pltpu.sync_copy(seq_hbm.at[pl.ds(h * B + b0, BPW)], idx_v)

    pltpu.sync_copy(pos_hbm.at[pl.ds(h * HALF, HALF)], bias_v)
    pltpu.sync_copy(seg_hbm, seg_v)
    s_first = 1 + 2 * h
    s_rest = 2 + 2 * h
    for grp in range(GROUPS):
        sl = pl.ds(grp * LANES, LANES)
        plsc.addupdate(bias_v.at[0, sl], seg_v[s_first, sl])

    @plsc.parallel_loop(1, HALF, unroll=4)
    def _(r):
        for grp in range(GROUPS):
            sl = pl.ds(grp * LANES, LANES)
            plsc.addupdate(bias_v.at[r, sl], seg_v[s_rest, sl])

    def start_gather(g, slot, sem):
        pltpu.async_copy(
            tok_hbm.at[idx_v.at[g // 2, pl.ds((g % 2) * C, C)]],
            buf_v.at[slot], sem)

    for g in range(NBUF - 1):
        start_gather(g, g, gsems[g])

    def ring_body(go, carry):
        for s in range(NBUF):
            g = go * NBUF + s
            nxt = (s + NBUF - 1) % NBUF
            pltpu.make_async_copy(
                tok_hbm.at[idx_v.at[0, pl.ds(0, C)]], buf_v.at[s],
                gsems[s]).wait()

            @pl.when(g >= 1)
            def _():
                pltpu.make_async_copy(
                    buf_v.at[nxt], out_hbm.at[pl.ds(0, C)],
                    osems[nxt]).wait()

            @pl.when(g + NBUF - 1 < GC)
            def _():
                start_gather(g + NBUF - 1, nxt, gsems[nxt])

            @plsc.parallel_loop(0, C, unroll=4)
            def _(r):
                for grp in range(GROUPS):
                    sl = pl.ds(grp * LANES, LANES)
                    plsc.addupdate(buf_v.at[s, r, sl],
                                   bias_v[(s % 2) * C + r, sl])

            row0 = (b0 + g // 2) * L + h * HALF + (s % 2) * C
            pltpu.async_copy(
                buf_v.at[s], out_hbm.at[pl.ds(row0, C)], osems[s])
        return carry

    lax.fori_loop(0, GC // NBUF, ring_body, 0)

    pltpu.make_async_copy(
        buf_v.at[NBUF - 1], out_hbm.at[pl.ds(0, C)],
        osems[NBUF - 1]).wait()


_sc_embed = functools.partial(
    pl.kernel,
    out_type=jax.ShapeDtypeStruct((BL, D), jnp.float32),
    mesh=plsc.VectorSubcoreMesh(core_axis_name="c", subcore_axis_name="s",
                                num_cores=NC, num_subcores=NS),
    scratch_types=[
        pltpu.VMEM((BPW, HALF), jnp.int32),
        pltpu.VMEM((NBUF, C, D), jnp.float32),
        pltpu.VMEM((HALF, D), jnp.float32),
        pltpu.VMEM((5, D), jnp.float32),
        [pltpu.SemaphoreType.DMA] * NBUF,
        [pltpu.SemaphoreType.DMA] * NBUF,
    ],
)(_sc_body)


def kernel(sequence, token_table, pos_table, seg_table):
    seq2 = sequence.reshape(B, 2, HALF).transpose(1, 0, 2).reshape(2 * B, HALF)
    pos_shifted = lax.slice_in_dim(pos_table, 1, L + 1, axis=0)
    out = _sc_embed(seq2, token_table, pos_shifted, seg_table)
    return out.reshape(B, L, D)


# speedup vs baseline: 1.0049x; 1.0049x over previous
import functools

import jax
import jax.numpy as jnp
from jax import lax
from jax.experimental import pallas as pl
from jax.experimental.pallas import tpu as pltpu
from jax.experimental.pallas import tpu_sc as plsc

D = 128
L = 512
CTX = 255
B = 1024
BL = B * L
NC, NS = 2, 16
NW = NC * NS
HALF = L // 2
BPW = B // (NW // 2)
C = 128
GC = BPW * HALF // C
NBUF = 2
LANES = 16
GROUPS = D // LANES


def _sc_body(seq_hbm, tok_hbm, pos_hbm, seg_hbm, out_hbm,
             idx_v, buf_v, bias_v, seg_v, gsems, osems):
    wid = lax.axis_index("s") * NC + lax.axis_index("c")
    h = wid % 2
    b0 = (wid // 2) * BPW

    pltpu.sync_copy(seq_hbm.at[pl.ds(h * B + b0, BPW)], idx_v)

    pltpu.sync_copy(pos_hbm.at[pl.ds(h * HALF, HALF)], bias_v)
    pltpu.sync_copy(seg_hbm, seg_v)
    s_first = 1 + 2 * h
    s_rest = 2 + 2 * h
    for grp in range(GROUPS):
        sl = pl.ds(grp * LANES, LANES)
        plsc.addupdate(bias_v.at[0, sl], seg_v[s_first, sl])

    @plsc.parallel_loop(1, HALF, unroll=4)
    def _(r):
        for grp in range(GROUPS):
            sl = pl.ds(grp * LANES, LANES)
            plsc.addupdate(bias_v.at[r, sl], seg_v[s_rest, sl])

    def start_gather(g, slot, sem):
        pltpu.async_copy(
            tok_hbm.at[idx_v.at[g // 2, pl.ds((g % 2) * C, C)]],
            buf_v.at[slot], sem)

    for g in range(NBUF - 1):
        start_gather(g, g, gsems[g])

    def ring_body(go, carry):
        for s in range(NBUF):
            g = go * NBUF + s
            nxt = (s + NBUF - 1) % NBUF
            pltpu.make_async_copy(
                tok_hbm.at[idx_v.at[0, pl.ds(0, C)]], buf_v.at[s],
                gsems[s]).wait()

            @pl.when(g >= 1)
            def _():
                pltpu.make_async_copy(
                    buf_v.at[nxt], out_hbm.at[pl.ds(0, C)],
                    osems[nxt]).wait()

            @pl.when(g + NBUF - 1 < GC)
            def _():
                start_gather(g + NBUF - 1, nxt, gsems[nxt])

            @plsc.parallel_loop(0, C, unroll=4)
            def _(r):
                for grp in range(GROUPS):
                    sl = pl.ds(grp * LANES, LANES)
                    plsc.addupdate(buf_v.at[s, r, sl],
                                   bias_v[(s % 2) * C + r, sl])

            row0 = (b0 + g // 2) * L + h * HALF + (s % 2) * C
            pltpu.async_copy(
                buf_v.at[s], out_hbm.at[pl.ds(row0, C)], osems[s])
        return carry

    lax.fori_loop(0, GC // NBUF, ring_body, 0)

    pltpu.make_async_copy(
        buf_v.at[NBUF - 1], out_hbm.at[pl.ds(0, C)],
        osems[NBUF - 1]).wait()


_sc_embed = functools.partial(
    pl.kernel,
    out_type=jax.ShapeDtypeStruct((BL, D), jnp.float32),
    mesh=plsc.VectorSubcoreMesh(core_axis_name="c", subcore_axis_name="s",
                                num_cores=NC, num_subcores=NS),
    scratch_types=[
        pltpu.VMEM((BPW, HALF), jnp.int32),
        pltpu.VMEM((NBUF, C, D), jnp.float32),
        pltpu.VMEM((HALF, D), jnp.float32),
        pltpu.VMEM((5, D), jnp.float32),
        [pltpu.SemaphoreType.DMA] * NBUF,
        [pltpu.SemaphoreType.DMA] * NBUF,
    ],
)(_sc_body)


def kernel(sequence, token_table, pos_table, seg_table):
    seq2 = sequence.reshape(B, 2, HALF).transpose(1, 0, 2).reshape(2 * B, HALF)
    pos_shifted = lax.slice_in_dim(pos_table, 1, L + 1, axis=0)
    out = _sc_embed(seq2, token_table, pos_shifted, seg_table)
    return out.reshape(B, L, D)
